# baseline (speedup 1.0000x reference)
import functools

import jax
import jax.numpy as jnp
import numpy as np
from jax.experimental import pallas as pl

_HIST = 50
_WIDTH = 74
_ROW_BLOCK = 4096
_PACK_BLOCK = 2048

_SHIFTS = np.zeros((_WIDTH,), np.int32)
_SHIFTS[24:31] = 8
_SHIFTS[31:62] = 16
_SHIFTS[62:74] = 24
_POS = np.arange(_WIDTH, dtype=np.int64)
_MASK_CONST = ((np.int64(0xFF) << _SHIFTS.astype(np.int64)) & 0xFFFFFFFF).astype(np.uint32).view(np.int32).reshape(1, _WIDTH)
_TARGET_CONST = ((_POS << _SHIFTS.astype(np.int64)) & 0xFFFFFFFF).astype(np.uint32).view(np.int32).reshape(1, _WIDTH)
_OFFSET_WORD = np.int32((24 << 8) | (31 << 16) | (62 << 24))


def _pack_body(h_ref, dw_ref, dm_ref, mo_ref, w_ref):
    w = (h_ref[...]
         + (dw_ref[...] << 8)
         + (dm_ref[...] << 16)
         + (mo_ref[...] << 24)
         + _OFFSET_WORD)
    w_ref[...] = w


def _expand_body(w_ref, o_ref):
    lane = jax.lax.broadcasted_iota(jnp.int32, (1, _WIDTH), 1)
    s = ((lane >= 24).astype(jnp.int32)
         + (lane >= 31).astype(jnp.int32)
         + (lane >= 62).astype(jnp.int32)) << 3
    mask_c = jnp.int32(0xFF) << s
    target_c = lane << s
    w = w_ref[...]
    wb = jnp.broadcast_to(w, (w.shape[0], _WIDTH))
    o_ref[...] = ((wb & mask_c) == target_c).astype(jnp.float32)


@functools.partial(jax.jit, static_argnums=())
def kernel(hour, day_of_week, day_of_month, month):
    b, hist = hour.shape
    rows = b * hist
    args = [x.astype(jnp.int32) for x in (hour, day_of_week, day_of_month, month)]

    pack_spec = pl.BlockSpec((_PACK_BLOCK, hist), lambda i: (i, 0))
    packed = pl.pallas_call(
        _pack_body,
        grid=(b // _PACK_BLOCK,),
        in_specs=[pack_spec] * 4,
        out_specs=pack_spec,
        out_shape=jax.ShapeDtypeStruct((b, hist), jnp.int32),
    )(*args)

    w_col = packed.reshape(rows, 1)
    out2d = pl.pallas_call(
        _expand_body,
        grid=(rows // _ROW_BLOCK,),
        in_specs=[pl.BlockSpec((_ROW_BLOCK, 1), lambda i: (i, 0))],
        out_specs=pl.BlockSpec((_ROW_BLOCK, _WIDTH), lambda i: (i, 0)),
        out_shape=jax.ShapeDtypeStruct((rows, _WIDTH), jnp.float32),
    )(w_col)
    return out2d.reshape(b, hist, _WIDTH)


# speedup vs baseline: 1.0712x; 1.0712x over previous
import functools

import jax
import jax.numpy as jnp
from jax.experimental import pallas as pl

_HIST = 50
_WIDTH = 74
_ROW_BLOCK = 4096
_UNROLL = _ROW_BLOCK // 8
_OFFSET_WORD = (24 << 8) | (31 << 16) | (62 << 24)


def _body(h_ref, dw_ref, dm_ref, mo_ref, o_ref):
    lane = jax.lax.broadcasted_iota(jnp.int32, (1, _WIDTH), 1)
    s = ((lane >= 24).astype(jnp.int32)
         + (lane >= 31).astype(jnp.int32)
         + (lane >= 62).astype(jnp.int32)) << 3
    mask_c = jnp.int32(0xFF) << s
    target_c = lane << s

    w8 = (h_ref[...]
          + (dw_ref[...] << 8)
          + (dm_ref[...] << 16)
          + (mo_ref[...] << 24)
          + jnp.int32(_OFFSET_WORD))
    wt = w8.T
    for v in range(_UNROLL):
        wcol = jax.lax.slice(wt, (0, v), (8, v + 1))
        wb = jnp.broadcast_to(wcol, (8, _WIDTH))
        o_ref[8 * v:8 * v + 8, :] = ((wb & mask_c) == target_c).astype(jnp.float32)


@functools.partial(jax.jit, static_argnums=())
def kernel(hour, day_of_week, day_of_month, month):
    b, hist = hour.shape
    rows = b * hist
    args = [x.astype(jnp.int32).reshape(rows // 8, 8)
            for x in (hour, day_of_week, day_of_month, month)]

    in_spec = pl.BlockSpec((_UNROLL, 8), lambda i: (i, 0))
    out_spec = pl.BlockSpec((_ROW_BLOCK, _WIDTH), lambda i: (i, 0))
    out2d = pl.pallas_call(
        _body,
        grid=(rows // _ROW_BLOCK,),
        in_specs=[in_spec] * 4,
        out_specs=out_spec,
        out_shape=jax.ShapeDtypeStruct((rows, _WIDTH), jnp.float32),
    )(*args)
    return out2d.reshape(b, hist, _WIDTH)


# speedup vs baseline: 2.0540x; 1.9174x over previous
import functools

import jax
import jax.numpy as jnp
from jax.experimental import pallas as pl

_HIST = 50
_WIDTH = 74
_BATCH_BLOCK = 128
_OFFSET_WORD = (24 << 8) | (31 << 16) | (62 << 24)


def _body(h_ref, dw_ref, dm_ref, mo_ref, o_ref):
    lane = jax.lax.broadcasted_iota(jnp.int32, (1, _WIDTH), 1)
    s = ((lane >= 24).astype(jnp.int32)
         + (lane >= 31).astype(jnp.int32)
         + (lane >= 62).astype(jnp.int32)) << 3
    mask_c = jnp.int32(0xFF) << s
    target_c = lane << s

    w = (h_ref[...]
         + (dw_ref[...] << 8)
         + (dm_ref[...] << 16)
         + (mo_ref[...] << 24)
         + jnp.int32(_OFFSET_WORD))
    wt = w.T
    for b in range(_BATCH_BLOCK):
        for j in range((_HIST + 7) // 8):
            lo = 8 * j
            hi = min(lo + 8, _HIST)
            wcol = jax.lax.slice(wt, (lo, b), (hi, b + 1))
            wb = jnp.broadcast_to(wcol, (hi - lo, _WIDTH))
            o_ref[b, lo:hi, :] = ((wb & mask_c) == target_c).astype(jnp.float32)


@functools.partial(jax.jit, static_argnums=())
def kernel(hour, day_of_week, day_of_month, month):
    b, hist = hour.shape
    args = [x.astype(jnp.int32) for x in (hour, day_of_week, day_of_month, month)]

    in_spec = pl.BlockSpec((_BATCH_BLOCK, hist), lambda i: (i, 0))
    out_spec = pl.BlockSpec((_BATCH_BLOCK, hist, _WIDTH), lambda i: (i, 0, 0))
    return pl.pallas_call(
        _body,
        grid=(b // _BATCH_BLOCK,),
        in_specs=[in_spec] * 4,
        out_specs=out_spec,
        out_shape=jax.ShapeDtypeStruct((b, hist, _WIDTH), jnp.float32),
    )(*args)


# speedup vs baseline: 14.7149x; 7.1639x over previous
import functools

import jax
import jax.numpy as jnp
from jax.experimental import pallas as pl

_HIST = 50
_WIDTH = 74
_BATCH_BLOCK = 512
_OFFSET_WORD = (24 << 8) | (31 << 16) | (62 << 24)


def _body(h_ref, dw_ref, dm_ref, mo_ref, o_ref):
    feat = jax.lax.broadcasted_iota(jnp.int32, (_WIDTH, 1), 0)
    s = ((feat >= 24).astype(jnp.int32)
         + (feat >= 31).astype(jnp.int32)
         + (feat >= 62).astype(jnp.int32)) << 3
    mask_c = jnp.int32(0xFF) << s
    target_c = feat << s

    w = (h_ref[...]
         + (dw_ref[...] << 8)
         + (dm_ref[...] << 16)
         + (mo_ref[...] << 24)
         + jnp.int32(_OFFSET_WORD))
    for h in range(_HIST):
        wrow = w[h:h + 1, :]
        o_ref[h] = ((wrow & mask_c) == target_c).astype(jnp.float32)


@functools.partial(jax.jit, static_argnums=())
def kernel(hour, day_of_week, day_of_month, month):
    b, hist = hour.shape
    args = [x.astype(jnp.int32).T for x in (hour, day_of_week, day_of_month, month)]

    in_spec = pl.BlockSpec((hist, _BATCH_BLOCK), lambda i: (0, i))
    out_spec = pl.BlockSpec((hist, _WIDTH, _BATCH_BLOCK), lambda i: (0, 0, i))
    out_t = pl.pallas_call(
        _body,
        grid=(b // _BATCH_BLOCK,),
        in_specs=[in_spec] * 4,
        out_specs=out_spec,
        out_shape=jax.ShapeDtypeStruct((hist, _WIDTH, b), jnp.float32),
    )(*args)
    return jnp.transpose(out_t, (2, 0, 1))


# speedup vs baseline: 14.7900x; 1.0051x over previous
import functools

import jax
import jax.numpy as jnp
from jax.experimental import pallas as pl

_HIST = 50
_WIDTH = 74
_BATCH_BLOCK = 1024
_OFFSET_WORD = (24 << 8) | (31 << 16) | (62 << 24)


def _body(h_ref, dw_ref, dm_ref, mo_ref, o_ref):
    feat = jax.lax.broadcasted_iota(jnp.int32, (_WIDTH, 1), 0)
    s = ((feat >= 24).astype(jnp.int32)
         + (feat >= 31).astype(jnp.int32)
         + (feat >= 62).astype(jnp.int32)) << 3
    mask_c = jnp.int32(0xFF) << s
    target_c = feat << s

    w = (h_ref[...]
         + (dw_ref[...] << 8)
         + (dm_ref[...] << 16)
         + (mo_ref[...] << 24)
         + jnp.int32(_OFFSET_WORD))
    for h in range(_HIST):
        wrow = w[h:h + 1, :]
        o_ref[h] = ((wrow & mask_c) == target_c).astype(jnp.float32)


@functools.partial(jax.jit, static_argnums=())
def kernel(hour, day_of_week, day_of_month, month):
    b, hist = hour.shape
    args = [x.astype(jnp.int32).T for x in (hour, day_of_week, day_of_month, month)]

    in_spec = pl.BlockSpec((hist, _BATCH_BLOCK), lambda i: (0, i))
    out_spec = pl.BlockSpec((hist, _WIDTH, _BATCH_BLOCK), lambda i: (0, 0, i))
    out_t = pl.pallas_call(
        _body,
        grid=(b // _BATCH_BLOCK,),
        in_specs=[in_spec] * 4,
        out_specs=out_spec,
        out_shape=jax.ShapeDtypeStruct((hist, _WIDTH, b), jnp.float32),
    )(*args)
    return jnp.transpose(out_t, (2, 0, 1))
